# baseline (speedup 1.0000x reference)
import jax
import jax.numpy as jnp
from jax import lax
from jax.experimental import pallas as pl
from jax.experimental.pallas import tpu as pltpu
from jax.experimental.pallas import tpu_sc as plsc

_SC_INFO = plsc.get_sparse_core_info()
_NC = _SC_INFO.num_cores
_NS = _SC_INFO.num_subcores
_NW = _NC * _NS

_SEQ, _D = 8192, 2048
_HALF = _SEQ // 2

_T_ROWS_PER_W = _HALF // _NW
_T_C = 24
_T_CHUNKS = []
_off = 0
while _off < _T_ROWS_PER_W:
    _sz = min(_T_C, _T_ROWS_PER_W - _off)
    _T_CHUNKS.append((_off, _sz))
    _off += _sz
_T_NCH = len(_T_CHUNKS)
_T_NBUF = 2


def _tec_body(enc_hbm, out_hbm, *scratch):
    bufs = scratch[:_T_NBUF]
    gsems = scratch[_T_NBUF:2 * _T_NBUF]
    ssems = scratch[2 * _T_NBUF:3 * _T_NBUF]
    wid = lax.axis_index("s") * _NC + lax.axis_index("c")
    base = wid * _T_ROWS_PER_W

    def start_gather(g):
        off, sz = _T_CHUNKS[g]
        return pltpu.async_copy(
            enc_hbm.at[pl.ds(base + off, sz)],
            bufs[g % _T_NBUF].at[pl.ds(0, sz)],
            gsems[g % _T_NBUF],
        )

    def start_scatter(g):
        off, sz = _T_CHUNKS[g]
        return pltpu.async_copy(
            bufs[g % _T_NBUF].at[pl.ds(0, sz)],
            out_hbm.at[pl.ds(base + off, sz)],
            ssems[g % _T_NBUF],
        )

    scat = [None] * _T_NBUF
    gat = [None] * _T_NBUF
    for g in range(min(_T_NBUF, _T_NCH)):
        gat[g % _T_NBUF] = start_gather(g)
    for g in range(_T_NCH):
        gat[g % _T_NBUF].wait()
        scat[g % _T_NBUF] = start_scatter(g)
        nxt = g + _T_NBUF
        if nxt < _T_NCH:
            scat[nxt % _T_NBUF].wait()
            gat[nxt % _T_NBUF] = start_gather(nxt)
            scat[nxt % _T_NBUF] = None
    for s in scat:
        if s is not None:
            s.wait()


_S_ROWS_PER_SC = _HALF // 2
_S_C = 128
_S_NCH = _S_ROWS_PER_SC // _S_C
_S_NBUF = 4
_S_LEAD = 2


def _scs_body(enc_hbm, out_hbm, *scratch):
    bufs = scratch[:_S_NBUF]
    gsems = scratch[_S_NBUF:2 * _S_NBUF]
    ssems = scratch[2 * _S_NBUF:3 * _S_NBUF]
    cid = lax.axis_index("c")
    base = _HALF + cid * _S_ROWS_PER_SC

    def start_gather(g):
        return pltpu.async_copy(
            enc_hbm.at[pl.ds(base + g * _S_C, _S_C)],
            bufs[g % _S_NBUF],
            gsems[g % _S_NBUF],
        )

    def start_scatter(g):
        return pltpu.async_copy(
            bufs[g % _S_NBUF],
            out_hbm.at[pl.ds(cid * _S_ROWS_PER_SC + g * _S_C, _S_C)],
            ssems[g % _S_NBUF],
        )

    gat = [None] * _S_NCH
    scat = [None] * _S_NCH
    for j in range(_S_LEAD):
        gat[j] = start_gather(j)
    for g in range(_S_NCH):
        j = g + _S_LEAD
        if j < _S_NCH:
            jn = j - _S_NBUF
            if jn >= 0:
                scat[jn].wait()
            gat[j] = start_gather(j)
        gat[g].wait()
        scat[g] = start_scatter(g)
    for g in range(max(0, _S_NCH - _S_NBUF), _S_NCH):
        scat[g].wait()


def kernel(x, encodings):
    seq, d = encodings.shape
    vmesh = plsc.VectorSubcoreMesh(core_axis_name="c", subcore_axis_name="s")
    top = pl.kernel(
        _tec_body,
        mesh=vmesh,
        out_type=jax.ShapeDtypeStruct((_HALF, d), jnp.float32),
        scratch_types=(
            [pltpu.VMEM((_T_C, _D), jnp.float32)] * _T_NBUF
            + [pltpu.SemaphoreType.DMA] * (2 * _T_NBUF)
        ),
    )(encodings)
    smesh = plsc.ScalarSubcoreMesh(axis_name="c", num_cores=2)
    bottom = pl.kernel(
        _scs_body,
        mesh=smesh,
        out_type=jax.ShapeDtypeStruct((_HALF, d), jnp.float32),
        scratch_types=(
            [pltpu.VMEM_SHARED((_S_C, _D), jnp.float32)] * _S_NBUF
            + [pltpu.SemaphoreType.DMA] * (2 * _S_NBUF)
        ),
    )(encodings)
    return top, bottom


# speedup vs baseline: 1.0785x; 1.0785x over previous
import jax
import jax.numpy as jnp
from jax import lax
from jax.experimental import pallas as pl
from jax.experimental.pallas import tpu as pltpu
from jax.experimental.pallas import tpu_sc as plsc

_SC_INFO = plsc.get_sparse_core_info()
_NC = _SC_INFO.num_cores
_NS = _SC_INFO.num_subcores
_NW = _NC * _NS


_SEQ, _D = 8192, 2048
_ROWS_PER_W = _SEQ // _NW
_C = 24
_CHUNKS = []
_off = 0
while _off < _ROWS_PER_W:
    _sz = min(_C, _ROWS_PER_W - _off)
    _CHUNKS.append((_off, _sz))
    _off += _sz
_NCH = len(_CHUNKS)
_NBUF = 2


def _sc_copy_body(enc_hbm, out_hbm, *scratch):
    bufs = scratch[:_NBUF]
    gsems = scratch[_NBUF:2 * _NBUF]
    ssems = scratch[2 * _NBUF:3 * _NBUF]
    wid = lax.axis_index("s") * _NC + lax.axis_index("c")
    base = wid * _ROWS_PER_W

    def start_gather(g):
        off, sz = _CHUNKS[g]
        return pltpu.async_copy(
            enc_hbm.at[pl.ds(base + off, sz)],
            bufs[g % _NBUF].at[pl.ds(0, sz)],
            gsems[g % _NBUF],
        )

    def start_scatter(g):
        off, sz = _CHUNKS[g]
        return pltpu.async_copy(
            bufs[g % _NBUF].at[pl.ds(0, sz)],
            out_hbm.at[pl.ds(base + off, sz)],
            ssems[g % _NBUF],
        )

    scat = [None] * _NBUF
    gat = [None] * _NBUF
    for g in range(min(_NBUF, _NCH)):
        gat[g % _NBUF] = start_gather(g)
    for g in range(_NCH):
        gat[g % _NBUF].wait()
        scat[g % _NBUF] = start_scatter(g)
        nxt = g + _NBUF
        if nxt < _NCH:
            scat[nxt % _NBUF].wait()
            gat[nxt % _NBUF] = start_gather(nxt)
            scat[nxt % _NBUF] = None
    for s in scat:
        if s is not None:
            s.wait()


def kernel(x, encodings):
    seq, d = encodings.shape
    mesh = plsc.VectorSubcoreMesh(core_axis_name="c", subcore_axis_name="s")
    out = pl.kernel(
        _sc_copy_body,
        mesh=mesh,
        out_type=jax.ShapeDtypeStruct((seq, d), jnp.float32),
        scratch_types=(
            [pltpu.VMEM((_C, _D), jnp.float32)] * _NBUF
            + [pltpu.SemaphoreType.DMA] * (2 * _NBUF)
        ),
    )(encodings)
    return out[None, :, :]
